# initial kernel scaffold (unmeasured)
import jax
import jax.numpy as jnp
from jax import lax
from jax.experimental import pallas as pl
from jax.experimental.pallas import tpu as pltpu

N_DEV = 32


def kernel(x, w_mat):
    k_total, k_per = x.shape
    _, n = w_mat.shape
    m_per = k_total // N_DEV

    def body(x_ref, w_ref, out_ref, xg_ref, amax_ref,
             send_sems, recv_sems, amax_send_sems, amax_recv_sems):
        my = lax.axis_index("i")

        barrier = pltpu.get_barrier_semaphore()
        for off in range(1, N_DEV):
            t = lax.rem(my + off, N_DEV)
            pl.semaphore_signal(barrier, inc=1, device_id=(t,),
                                device_id_type=pl.DeviceIdType.MESH)
        pl.semaphore_wait(barrier, N_DEV - 1)

        xg_ref[:, pl.ds(my * k_per, k_per)] = x_ref[pl.ds(my * m_per, m_per), :]

        rdmas = []
        for off in range(1, N_DEV):
            t = lax.rem(my + off, N_DEV)
            rdma = pltpu.make_async_remote_copy(
                src_ref=x_ref.at[pl.ds(t * m_per, m_per), :],
                dst_ref=xg_ref.at[:, pl.ds(my * k_per, k_per)],
                send_sem=send_sems.at[off - 1],
                recv_sem=recv_sems.at[off - 1],
                device_id=(t,),
                device_id_type=pl.DeviceIdType.MESH,
            )
            rdma.start()
            rdmas.append(rdma)
        for rdma in rdmas:
            rdma.wait()

        y = lax.dot_general(
            xg_ref[:, :], w_ref[:, :],
            dimension_numbers=(((1,), (0,)), ((), ())),
            precision=lax.Precision.HIGHEST,
            preferred_element_type=jnp.float32,
        )

        local_amax = jnp.max(jnp.abs(y))
        amax_ref[pl.ds(my, 1)] = jnp.full((1, 8, 128), local_amax, jnp.float32)
        amax_rdmas = []
        for off in range(1, N_DEV):
            t = lax.rem(my + off, N_DEV)
            rdma = pltpu.make_async_remote_copy(
                src_ref=amax_ref.at[pl.ds(my, 1)],
                dst_ref=amax_ref.at[pl.ds(my, 1)],
                send_sem=amax_send_sems.at[off - 1],
                recv_sem=amax_recv_sems.at[off - 1],
                device_id=(t,),
                device_id_type=pl.DeviceIdType.MESH,
            )
            rdma.start()
            amax_rdmas.append(rdma)
        for rdma in amax_rdmas:
            rdma.wait()
        g_amax = jnp.max(amax_ref[...])

        scale = g_amax / 448.0
        q = (y / scale).astype(jnp.float8_e4m3fn)
        out_ref[...] = q.astype(jnp.float32) * scale

    return pl.pallas_call(
        body,
        out_shape=jax.ShapeDtypeStruct((m_per, n), jnp.float32),
        in_specs=[
            pl.BlockSpec(memory_space=pltpu.VMEM),
            pl.BlockSpec(memory_space=pltpu.VMEM),
        ],
        out_specs=pl.BlockSpec(memory_space=pltpu.VMEM),
        scratch_shapes=[
            pltpu.VMEM((m_per, k_total), jnp.float32),
            pltpu.VMEM((N_DEV, 8, 128), jnp.float32),
            pltpu.SemaphoreType.DMA((N_DEV - 1,)),
            pltpu.SemaphoreType.DMA((N_DEV - 1,)),
            pltpu.SemaphoreType.DMA((N_DEV - 1,)),
            pltpu.SemaphoreType.DMA((N_DEV - 1,)),
        ],
        compiler_params=pltpu.CompilerParams(collective_id=0),
    )(x, w_mat)


# baseline (device time: 63105 ns/iter reference)
import os

import jax
import jax.numpy as jnp
from jax import lax
from jax.experimental import pallas as pl
from jax.experimental.pallas import tpu as pltpu

N_DEV = 32
_STAGE = int(os.environ.get("KSTAGE", "3"))


def kernel(x, w_mat):
    k_total, k_per = x.shape
    _, n = w_mat.shape
    m_per = k_total // N_DEV

    K_CHUNK = 512
    n_chunks = k_total // K_CHUNK

    def body(x_ref, w_ref, out_ref, xg_ref, wbuf_ref, amax_ref,
             send_sems, recv_sems, amax_send_sems, amax_recv_sems, wdma_sems):
        my = lax.axis_index("i")

        if _STAGE >= 1:
            barrier = pltpu.get_barrier_semaphore()
            for off in range(1, N_DEV):
                t = lax.rem(my + off, N_DEV)
                pl.semaphore_signal(barrier, inc=1, device_id=(t,),
                                    device_id_type=pl.DeviceIdType.MESH)
            pl.semaphore_wait(barrier, N_DEV - 1)

        xg_ref[:, pl.ds(my * k_per, k_per)] = x_ref[pl.ds(my * m_per, m_per), :]

        rdmas = []
        if _STAGE >= 2:
            for off in range(1, N_DEV):
                t = lax.rem(my + off, N_DEV)
                rdma = pltpu.make_async_remote_copy(
                    src_ref=x_ref.at[pl.ds(t * m_per, m_per), :],
                    dst_ref=xg_ref.at[:, pl.ds(my * k_per, k_per)],
                    send_sem=send_sems.at[off - 1],
                    recv_sem=recv_sems.at[off - 1],
                    device_id=(t,),
                    device_id_type=pl.DeviceIdType.MESH,
                )
                rdma.start()
                rdmas.append(rdma)
        def w_dma(c):
            return pltpu.make_async_copy(
                w_ref.at[pl.ds(c * K_CHUNK, K_CHUNK), :],
                wbuf_ref.at[c % 2],
                wdma_sems.at[c % 2],
            )

        w_dma(0).start()
        for rdma in rdmas:
            rdma.wait()

        y = jnp.zeros((m_per, n), jnp.float32)
        for c in range(n_chunks):
            if c + 1 < n_chunks:
                w_dma(c + 1).start()
            w_dma(c).wait()
            y = y + lax.dot_general(
                xg_ref[:, pl.ds(c * K_CHUNK, K_CHUNK)], wbuf_ref[c % 2],
                dimension_numbers=(((1,), (0,)), ((), ())),
                precision=lax.Precision.HIGHEST,
                preferred_element_type=jnp.float32,
            )

        local_amax = jnp.max(jnp.abs(y))
        amax_ref[pl.ds(my, 1)] = jnp.full((1, 8, 128), local_amax, jnp.float32)
        if _STAGE >= 3:
            amax_rdmas = []
            for off in range(1, N_DEV):
                t = lax.rem(my + off, N_DEV)
                rdma = pltpu.make_async_remote_copy(
                    src_ref=amax_ref.at[pl.ds(my, 1)],
                    dst_ref=amax_ref.at[pl.ds(my, 1)],
                    send_sem=amax_send_sems.at[off - 1],
                    recv_sem=amax_recv_sems.at[off - 1],
                    device_id=(t,),
                    device_id_type=pl.DeviceIdType.MESH,
                )
                rdma.start()
                amax_rdmas.append(rdma)
            for rdma in amax_rdmas:
                rdma.wait()
            g_amax = jnp.max(amax_ref[...])
        else:
            g_amax = local_amax

        scale = g_amax / 448.0
        q = (y / scale).astype(jnp.float8_e4m3fn)
        out_ref[...] = q.astype(jnp.float32) * scale

    return pl.pallas_call(
        body,
        out_shape=jax.ShapeDtypeStruct((m_per, n), jnp.float32),
        in_specs=[
            pl.BlockSpec(memory_space=pltpu.VMEM),
            pl.BlockSpec(memory_space=pl.ANY),
        ],
        out_specs=pl.BlockSpec(memory_space=pltpu.VMEM),
        scratch_shapes=[
            pltpu.VMEM((m_per, k_total), jnp.float32),
            pltpu.VMEM((2, K_CHUNK, n), jnp.float32),
            pltpu.VMEM((N_DEV, 8, 128), jnp.float32),
            pltpu.SemaphoreType.DMA((N_DEV - 1,)),
            pltpu.SemaphoreType.DMA((N_DEV - 1,)),
            pltpu.SemaphoreType.DMA((N_DEV - 1,)),
            pltpu.SemaphoreType.DMA((N_DEV - 1,)),
            pltpu.SemaphoreType.DMA((2,)),
        ],
        compiler_params=pltpu.CompilerParams(collective_id=0),
    )(x, w_mat)
